# Initial kernel scaffold; baseline (speedup 1.0000x reference)
import jax
import jax.numpy as jnp
from jax.experimental import pallas as pl


def kernel(x, w, b, alpha, gamma, beta):
    raise NotImplementedError("write your pallas kernel here")


# speedup vs baseline: 54.3485x; 54.3485x over previous
import functools

import jax
import jax.numpy as jnp
from jax.experimental import pallas as pl
from jax.experimental.pallas import tpu as pltpu

_BN_EPS = 1e-5
_CHUNK = 16


def _conv_block(xp_ref, w_ref, b_ref, alpha_ref, dd, h):
    bn_ = xp_ref.shape[2]
    wci = xp_ref.shape[3]
    wc = w_ref.shape[3]
    acc = jnp.zeros((h * bn_, wc), jnp.float32)
    for kd in range(3):
        for kh in range(3):
            patch = xp_ref[dd + kd, kh:kh + h].reshape(h * bn_, wci)
            acc = acc + jnp.dot(patch, w_ref[kd, kh],
                                preferred_element_type=jnp.float32)
    y = acc + b_ref[...]
    return jnp.where(y >= 0.0, y, alpha_ref[...] * y)


def _stats_kernel(xp_ref, w_ref, b_ref, alpha_ref, st_ref, *, d, h):
    wc = w_ref.shape[3]
    s1 = jnp.zeros((1, wc), jnp.float32)
    s2 = jnp.zeros((1, wc), jnp.float32)
    for dd in range(d):
        y = _conv_block(xp_ref, w_ref, b_ref, alpha_ref, dd, h)
        s1 = s1 + jnp.sum(y, axis=0, keepdims=True)
        s2 = s2 + jnp.sum(y * y, axis=0, keepdims=True)
    st_ref[0, 0:1] = s1
    st_ref[0, 1:2] = s2


def _scale_shift_kernel(part_ref, fold_ref, gam_ref, bet_ref, ss_ref, *, inv_count):
    sums = jnp.sum(part_ref[...], axis=0)
    tot = jnp.dot(sums, fold_ref[...], preferred_element_type=jnp.float32)
    mean = tot[0:1, :] * inv_count
    var = tot[1:2, :] * inv_count - mean * mean
    scale = jax.lax.rsqrt(var + _BN_EPS) * gam_ref[...]
    shift = bet_ref[...] - mean * scale
    ss_ref[0:1, :] = scale
    ss_ref[1:2, :] = shift


def _apply_kernel(xp_ref, w_ref, b_ref, alpha_ref, ss_ref, o_ref, *, d, h, w):
    bn_ = xp_ref.shape[2]
    for dd in range(d):
        y = _conv_block(xp_ref, w_ref, b_ref, alpha_ref, dd, h)
        y = y * ss_ref[0:1, :] + ss_ref[1:2, :]
        o_ref[dd] = y.reshape(h, bn_, y.shape[1]).astype(o_ref.dtype)


@jax.jit
def _conv_prelu_bn(x, w, b, alpha, gamma, beta):
    N, Cin, D, H, W = x.shape
    Cout = w.shape[0]
    WC = W * Cout
    WCI = (W + 2) * Cin
    M = N * D * H * W

    x_t = jnp.transpose(x, (2, 3, 0, 4, 1)).astype(jnp.bfloat16)
    xp = jnp.pad(x_t, ((1, 1), (1, 1), (0, 0), (1, 1), (0, 0)))
    xp = xp.reshape(D + 2, H + 2, N, WCI)

    wt = jnp.transpose(w, (2, 3, 4, 1, 0)).astype(jnp.float32)
    shift = jnp.stack([jnp.pad(jnp.eye(W, dtype=jnp.float32), ((kw, 2 - kw), (0, 0)))
                       for kw in range(3)])
    bweight = jnp.einsum("kpw,dhkio->dhpiwo", shift, wt).reshape(3, 3, WCI, WC)
    bweight = bweight.astype(jnp.bfloat16)

    b_t = jnp.tile(b.astype(jnp.float32), W).reshape(1, WC)
    alpha_t = jnp.reshape(alpha, (1, 1)).astype(jnp.float32)
    gam_t = jnp.tile(gamma.astype(jnp.float32), W).reshape(1, WC)
    bet_t = jnp.tile(beta.astype(jnp.float32), W).reshape(1, WC)
    fold = jnp.tile(jnp.eye(Cout, dtype=jnp.float32), (W, W))

    bn_ = _CHUNK
    while N % bn_:
        bn_ //= 2
    P = N // bn_

    xp_spec = pl.BlockSpec((D + 2, H + 2, bn_, WCI), lambda i: (0, 0, i, 0))
    w_spec = pl.BlockSpec((3, 3, WCI, WC), lambda i: (0, 0, 0, 0))
    row_spec = pl.BlockSpec((1, WC), lambda i: (0, 0))
    a_spec = pl.BlockSpec((1, 1), lambda i: (0, 0))
    par = pltpu.CompilerParams(dimension_semantics=("parallel",))

    partials = pl.pallas_call(
        functools.partial(_stats_kernel, d=D, h=H),
        grid=(P,),
        in_specs=[xp_spec, w_spec, row_spec, a_spec],
        out_specs=pl.BlockSpec((1, 2, WC), lambda i: (i, 0, 0)),
        out_shape=jax.ShapeDtypeStruct((P, 2, WC), jnp.float32),
        compiler_params=par,
    )(xp, bweight, b_t, alpha_t)

    scale_shift = pl.pallas_call(
        functools.partial(_scale_shift_kernel, inv_count=1.0 / M),
        grid=(1,),
        in_specs=[
            pl.BlockSpec((P, 2, WC), lambda i: (0, 0, 0)),
            pl.BlockSpec((WC, WC), lambda i: (0, 0)),
            row_spec,
            row_spec,
        ],
        out_specs=pl.BlockSpec((2, WC), lambda i: (0, 0)),
        out_shape=jax.ShapeDtypeStruct((2, WC), jnp.float32),
    )(partials, fold, gam_t, bet_t)

    out_t = pl.pallas_call(
        functools.partial(_apply_kernel, d=D, h=H, w=W),
        grid=(P,),
        in_specs=[xp_spec, w_spec, row_spec, a_spec,
                  pl.BlockSpec((2, WC), lambda i: (0, 0))],
        out_specs=pl.BlockSpec((D, H, bn_, WC), lambda i: (0, 0, i, 0)),
        out_shape=jax.ShapeDtypeStruct((D, H, N, WC), jnp.bfloat16),
        compiler_params=par,
    )(xp, bweight, b_t, alpha_t, scale_shift)

    out = out_t.reshape(D, H, N, W, Cout)
    return jnp.transpose(out, (2, 4, 0, 1, 3)).astype(jnp.float32)


def kernel(x, w, b, alpha, gamma, beta):
    return _conv_prelu_bn(x, w, b, alpha, gamma, beta)


# speedup vs baseline: 61.8457x; 1.1379x over previous
import functools

import jax
import jax.numpy as jnp
from jax.experimental import pallas as pl
from jax.experimental.pallas import tpu as pltpu

_BN_EPS = 1e-5
_CHUNK = 16


def _conv_block(xp_ref, w_ref, b_ref, alpha_ref, dd, d, h):
    bn_ = xp_ref.shape[2]
    wci = xp_ref.shape[3]
    wc = w_ref.shape[3]
    hb = h * bn_
    acc = jnp.zeros((hb, wc), jnp.float32)
    for kd in range(3):
        p = dd + kd - 1
        if p < 0 or p >= d:
            continue
        lhs = xp_ref[p].reshape(hb, wci)
        for kh in range(3):
            res = jnp.dot(lhs, w_ref[kd, kh], preferred_element_type=jnp.float32)
            zrow = jnp.zeros((bn_, wc), jnp.float32)
            if kh == 1:
                acc = acc + res
            elif kh == 0:
                acc = acc + jnp.concatenate([zrow, res[:hb - bn_, :]], axis=0)
            else:
                acc = acc + jnp.concatenate([res[bn_:, :], zrow], axis=0)
    y = acc + b_ref[...]
    return jnp.where(y >= 0.0, y, alpha_ref[...] * y)


def _stats_kernel(xp_ref, w_ref, b_ref, alpha_ref, st_ref, *, d, h):
    wc = w_ref.shape[3]
    s1 = jnp.zeros((1, wc), jnp.float32)
    s2 = jnp.zeros((1, wc), jnp.float32)
    for dd in range(d):
        y = _conv_block(xp_ref, w_ref, b_ref, alpha_ref, dd, d, h)
        s1 = s1 + jnp.sum(y, axis=0, keepdims=True)
        s2 = s2 + jnp.sum(y * y, axis=0, keepdims=True)
    st_ref[0, 0:1] = s1
    st_ref[0, 1:2] = s2


def _scale_shift_kernel(part_ref, fold_ref, gam_ref, bet_ref, ss_ref, *, inv_count):
    sums = jnp.sum(part_ref[...], axis=0)
    tot = jnp.dot(sums, fold_ref[...], preferred_element_type=jnp.float32)
    mean = tot[0:1, :] * inv_count
    var = tot[1:2, :] * inv_count - mean * mean
    scale = jax.lax.rsqrt(var + _BN_EPS) * gam_ref[...]
    shift = bet_ref[...] - mean * scale
    ss_ref[0:1, :] = scale
    ss_ref[1:2, :] = shift


def _apply_kernel(xp_ref, w_ref, b_ref, alpha_ref, ss_ref, o_ref, *, d, h):
    bn_ = xp_ref.shape[2]
    for dd in range(d):
        y = _conv_block(xp_ref, w_ref, b_ref, alpha_ref, dd, d, h)
        y = y * ss_ref[0:1, :] + ss_ref[1:2, :]
        o_ref[dd] = y.reshape(h, bn_, y.shape[1]).astype(o_ref.dtype)


@jax.jit
def _conv_prelu_bn(x, w, b, alpha, gamma, beta):
    N, Cin, D, H, W = x.shape
    Cout = w.shape[0]
    WC = W * Cout
    WCI = W * Cin
    M = N * D * H * W

    xp = jnp.transpose(x, (2, 3, 0, 4, 1)).astype(jnp.bfloat16).reshape(D, H, N, WCI)

    wt = jnp.transpose(w, (2, 3, 4, 1, 0)).astype(jnp.bfloat16)
    wcol = jnp.arange(W)
    band = []
    for kw in range(3):
        rows = wcol + kw - 1
        valid = (rows >= 0) & (rows < W)
        onehot = (rows[None, :] == wcol[:, None]) & valid[None, :]
        band.append(onehot.astype(jnp.bfloat16))
    shift = jnp.stack(band)
    bweight = jnp.einsum("kpw,dhkio->dhpiwo", shift, wt).reshape(3, 3, WCI, WC)
    bweight = bweight.astype(jnp.bfloat16)

    b_t = jnp.tile(b.astype(jnp.float32), W).reshape(1, WC)
    alpha_t = jnp.reshape(alpha, (1, 1)).astype(jnp.float32)
    gam_t = jnp.tile(gamma.astype(jnp.float32), W).reshape(1, WC)
    bet_t = jnp.tile(beta.astype(jnp.float32), W).reshape(1, WC)
    fold = jnp.tile(jnp.eye(Cout, dtype=jnp.float32), (W, W))

    bn_ = _CHUNK
    while N % bn_:
        bn_ //= 2
    P = N // bn_

    xp_spec = pl.BlockSpec((D, H, bn_, WCI), lambda i: (0, 0, i, 0))
    w_spec = pl.BlockSpec((3, 3, WCI, WC), lambda i: (0, 0, 0, 0))
    row_spec = pl.BlockSpec((1, WC), lambda i: (0, 0))
    a_spec = pl.BlockSpec((1, 1), lambda i: (0, 0))
    par = pltpu.CompilerParams(dimension_semantics=("arbitrary",))

    partials = pl.pallas_call(
        functools.partial(_stats_kernel, d=D, h=H),
        grid=(P,),
        in_specs=[xp_spec, w_spec, row_spec, a_spec],
        out_specs=pl.BlockSpec((1, 2, WC), lambda i: (i, 0, 0)),
        out_shape=jax.ShapeDtypeStruct((P, 2, WC), jnp.float32),
        compiler_params=par,
    )(xp, bweight, b_t, alpha_t)

    scale_shift = pl.pallas_call(
        functools.partial(_scale_shift_kernel, inv_count=1.0 / M),
        grid=(1,),
        in_specs=[
            pl.BlockSpec((P, 2, WC), lambda i: (0, 0, 0)),
            pl.BlockSpec((WC, WC), lambda i: (0, 0)),
            row_spec,
            row_spec,
        ],
        out_specs=pl.BlockSpec((2, WC), lambda i: (0, 0)),
        out_shape=jax.ShapeDtypeStruct((2, WC), jnp.float32),
    )(partials, fold, gam_t, bet_t)

    out_t = pl.pallas_call(
        functools.partial(_apply_kernel, d=D, h=H),
        grid=(P,),
        in_specs=[xp_spec, w_spec, row_spec, a_spec,
                  pl.BlockSpec((2, WC), lambda i: (0, 0))],
        out_specs=pl.BlockSpec((D, H, bn_, WC), lambda i: (0, 0, i, 0)),
        out_shape=jax.ShapeDtypeStruct((D, H, N, WC), jnp.bfloat16),
        compiler_params=par,
    )(xp, bweight, b_t, alpha_t, scale_shift)

    out = out_t.reshape(D, H, N, W, Cout)
    return jnp.transpose(out, (2, 4, 0, 1, 3)).astype(jnp.float32)


def kernel(x, w, b, alpha, gamma, beta):
    return _conv_prelu_bn(x, w, b, alpha, gamma, beta)


# speedup vs baseline: 62.2432x; 1.0064x over previous
import functools

import jax
import jax.numpy as jnp
from jax.experimental import pallas as pl
from jax.experimental.pallas import tpu as pltpu

_BN_EPS = 1e-5
_CHUNK = 16
_NCHUNK = 2


def _conv_block(xp_ref, w_ref, b_ref, alpha_ref, dd, c, ch, d, h):
    wci = xp_ref.shape[3]
    wc = w_ref.shape[3]
    hb = h * ch
    acc = jnp.zeros((hb, wc), jnp.float32)
    for kd in range(3):
        p = dd + kd - 1
        if p < 0 or p >= d:
            continue
        lhs = xp_ref[p, :, c * ch:(c + 1) * ch, :].reshape(hb, wci)
        for kh in range(3):
            res = jnp.dot(lhs, w_ref[kd, kh], preferred_element_type=jnp.float32)
            zrow = jnp.zeros((ch, wc), jnp.float32)
            if kh == 1:
                acc = acc + res
            elif kh == 0:
                acc = acc + jnp.concatenate([zrow, res[:hb - ch, :]], axis=0)
            else:
                acc = acc + jnp.concatenate([res[ch:, :], zrow], axis=0)
    y = acc + b_ref[...]
    return jnp.where(y >= 0.0, y, alpha_ref[...] * y)


def _stats_kernel(xp_ref, w_ref, b_ref, alpha_ref, st_ref, *, d, h):
    wc = w_ref.shape[3]
    s1 = jnp.zeros((1, wc), jnp.float32)
    s2 = jnp.zeros((1, wc), jnp.float32)
    bnblk = xp_ref.shape[2]
    ch = min(_CHUNK, bnblk)
    for c in range(bnblk // ch):
        for dd in range(d):
            y = _conv_block(xp_ref, w_ref, b_ref, alpha_ref, dd, c, ch, d, h)
            s1 = s1 + jnp.sum(y, axis=0, keepdims=True)
            s2 = s2 + jnp.sum(y * y, axis=0, keepdims=True)
    st_ref[0, 0:1] = s1
    st_ref[0, 1:2] = s2


def _scale_shift_kernel(part_ref, fold_ref, gam_ref, bet_ref, ss_ref, *, inv_count):
    sums = jnp.sum(part_ref[...], axis=0)
    tot = jnp.dot(sums, fold_ref[...], preferred_element_type=jnp.float32)
    mean = tot[0:1, :] * inv_count
    var = tot[1:2, :] * inv_count - mean * mean
    scale = jax.lax.rsqrt(var + _BN_EPS) * gam_ref[...]
    shift = bet_ref[...] - mean * scale
    ss_ref[0:1, :] = scale
    ss_ref[1:2, :] = shift


def _apply_kernel(xp_ref, w_ref, b_ref, alpha_ref, ss_ref, o_ref, *, d, h):
    bnblk = xp_ref.shape[2]
    ch = min(_CHUNK, bnblk)
    for c in range(bnblk // ch):
        for dd in range(d):
            y = _conv_block(xp_ref, w_ref, b_ref, alpha_ref, dd, c, ch, d, h)
            y = y * ss_ref[0:1, :] + ss_ref[1:2, :]
            o_ref[dd, :, c * ch:(c + 1) * ch, :] = (
                y.reshape(h, ch, y.shape[1]).astype(o_ref.dtype))


@jax.jit
def _conv_prelu_bn(x, w, b, alpha, gamma, beta):
    N, Cin, D, H, W = x.shape
    Cout = w.shape[0]
    WC = W * Cout
    WCI = W * Cin
    M = N * D * H * W

    xp = jnp.transpose(x, (2, 3, 0, 4, 1)).astype(jnp.bfloat16).reshape(D, H, N, WCI)

    wt = jnp.transpose(w, (2, 3, 4, 1, 0)).astype(jnp.bfloat16)
    wcol = jnp.arange(W)
    band = []
    for kw in range(3):
        rows = wcol + kw - 1
        valid = (rows >= 0) & (rows < W)
        onehot = (rows[None, :] == wcol[:, None]) & valid[None, :]
        band.append(onehot.astype(jnp.bfloat16))
    shift = jnp.stack(band)
    bweight = jnp.einsum("kpw,dhkio->dhpiwo", shift, wt).reshape(3, 3, WCI, WC)
    bweight = bweight.astype(jnp.bfloat16)

    b_t = jnp.tile(b.astype(jnp.float32), W).reshape(1, WC)
    alpha_t = jnp.reshape(alpha, (1, 1)).astype(jnp.float32)
    gam_t = jnp.tile(gamma.astype(jnp.float32), W).reshape(1, WC)
    bet_t = jnp.tile(beta.astype(jnp.float32), W).reshape(1, WC)
    fold = jnp.tile(jnp.eye(Cout, dtype=jnp.float32), (W, W))

    bw_ = _CHUNK * _NCHUNK
    while N % bw_:
        bw_ //= 2
    P = N // bw_

    xp_spec = pl.BlockSpec((D, H, bw_, WCI), lambda i: (0, 0, i, 0))
    w_spec = pl.BlockSpec((3, 3, WCI, WC), lambda i: (0, 0, 0, 0))
    row_spec = pl.BlockSpec((1, WC), lambda i: (0, 0))
    a_spec = pl.BlockSpec((1, 1), lambda i: (0, 0))
    par = pltpu.CompilerParams(dimension_semantics=("arbitrary",))

    partials = pl.pallas_call(
        functools.partial(_stats_kernel, d=D, h=H),
        grid=(P,),
        in_specs=[xp_spec, w_spec, row_spec, a_spec],
        out_specs=pl.BlockSpec((1, 2, WC), lambda i: (i, 0, 0)),
        out_shape=jax.ShapeDtypeStruct((P, 2, WC), jnp.float32),
        compiler_params=par,
    )(xp, bweight, b_t, alpha_t)

    scale_shift = pl.pallas_call(
        functools.partial(_scale_shift_kernel, inv_count=1.0 / M),
        grid=(1,),
        in_specs=[
            pl.BlockSpec((P, 2, WC), lambda i: (0, 0, 0)),
            pl.BlockSpec((WC, WC), lambda i: (0, 0)),
            row_spec,
            row_spec,
        ],
        out_specs=pl.BlockSpec((2, WC), lambda i: (0, 0)),
        out_shape=jax.ShapeDtypeStruct((2, WC), jnp.float32),
    )(partials, fold, gam_t, bet_t)

    out_t = pl.pallas_call(
        functools.partial(_apply_kernel, d=D, h=H),
        grid=(P,),
        in_specs=[xp_spec, w_spec, row_spec, a_spec,
                  pl.BlockSpec((2, WC), lambda i: (0, 0))],
        out_specs=pl.BlockSpec((D, H, bw_, WC), lambda i: (0, 0, i, 0)),
        out_shape=jax.ShapeDtypeStruct((D, H, N, WC), jnp.bfloat16),
        compiler_params=par,
    )(xp, bweight, b_t, alpha_t, scale_shift)

    out = out_t.reshape(D, H, N, W, Cout)
    return jnp.transpose(out, (2, 4, 0, 1, 3)).astype(jnp.float32)


def kernel(x, w, b, alpha, gamma, beta):
    return _conv_prelu_bn(x, w, b, alpha, gamma, beta)
